# baseline (speedup 1.0000x reference)
import functools

import jax
import jax.numpy as jnp
from jax import lax
from jax.experimental import pallas as pl
from jax.experimental.pallas import tpu as pltpu
from jax.experimental.pallas import tpu_sc as plsc

_LANES = 16


def _sc_dots(x, y, neg_flat, WI, WO, seq_table, T, *, B, K, D, NC, NS):
    NW = NC * NS
    Bw = B // NW
    C = 16
    n_chunks = Bw // C
    JV = D // _LANES

    mesh = plsc.VectorSubcoreMesh(core_axis_name="c", subcore_axis_name="s")

    @functools.partial(
        pl.kernel,
        mesh=mesh,
        compiler_params=pltpu.CompilerParams(needs_layout_passes=False),
        out_type=[
            jax.ShapeDtypeStruct((B,), jnp.float32),
            jax.ShapeDtypeStruct((NW, K, Bw), jnp.float32),
        ],
        scratch_types=[
            pltpu.VMEM((C,), jnp.int32),
            pltpu.VMEM((C,), jnp.int32),
            pltpu.VMEM((C * K,), jnp.int32),
            pltpu.VMEM((C, D), jnp.float32),
            pltpu.VMEM((C, D), jnp.float32),
            pltpu.VMEM((C, D), jnp.float32),
            pltpu.VMEM((C * K, D), jnp.float32),
            pltpu.VMEM((C * K, D), jnp.float32),
            pltpu.VMEM((C, D), jnp.float32),
            pltpu.VMEM((D,), jnp.float32),
            pltpu.VMEM((Bw,), jnp.float32),
            pltpu.VMEM((K, Bw), jnp.float32),
            pltpu.SemaphoreType.DMA,
        ],
    )
    def run(x_h, y_h, neg_h, wi_h, wo_h, seq_h, t_h, pos_h, negd_h,
            ix_v, iy_v, in_v, vi_v, woy_v, seqy_v, won_v, seqn_v,
            vit_v, t_v, pos_o, neg_o, sem):
        wid = lax.axis_index("s") * NC + lax.axis_index("c")
        base = wid * Bw
        pltpu.sync_copy(t_h, t_v)
        tj = [t_v[pl.ds(j * _LANES, _LANES)] for j in range(JV)]
        lane = lax.broadcasted_iota(jnp.int32, (_LANES,), 0)
        rowk = [lane * K + k for k in range(K)]

        def chunk_body(c, carry):
            tok0 = base + c * C
            pltpu.sync_copy(x_h.at[pl.ds(tok0, C)], ix_v)
            pltpu.sync_copy(y_h.at[pl.ds(tok0, C)], iy_v)
            pltpu.sync_copy(neg_h.at[pl.ds(tok0 * K, C * K)], in_v)
            cps = [
                pltpu.async_copy(wi_h.at[ix_v], vi_v, sem),
                pltpu.async_copy(wo_h.at[iy_v], woy_v, sem),
                pltpu.async_copy(seq_h.at[iy_v], seqy_v, sem),
                pltpu.async_copy(wo_h.at[in_v], won_v, sem),
                pltpu.async_copy(seq_h.at[in_v], seqn_v, sem),
            ]
            for cp in cps:
                cp.wait()

            def pre_body(g, acc):
                for j in range(JV):
                    sl = pl.ds(j * _LANES, _LANES)
                    vit_v[g, sl] = vi_v[g, sl] * tj[j]
                return acc
            lax.fori_loop(0, C, pre_body, 0)

            def d_body(i, accs):
                for dd in range(4):
                    d = i * 4 + dd
                    col = jnp.zeros((_LANES,), jnp.int32) + d
                    vi_c = plsc.load_gather(vi_v, [lane, col])
                    vit_c = plsc.load_gather(vit_v, [lane, col])
                    a0 = (accs[0]
                          + vi_c * plsc.load_gather(woy_v, [lane, col])
                          + vit_c * plsc.load_gather(seqy_v, [lane, col]))
                    new = [a0]
                    for k in range(K):
                        new.append(accs[k + 1]
                                   + vi_c * plsc.load_gather(won_v, [rowk[k], col])
                                   + vit_c * plsc.load_gather(seqn_v, [rowk[k], col]))
                    accs = tuple(new)
                return accs

            init = tuple(jnp.zeros((_LANES,), jnp.float32) for _ in range(K + 1))
            res = lax.fori_loop(0, D // 4, d_body, init)
            pos_o[pl.ds(c * C, C)] = res[0]
            for k in range(K):
                neg_o[k, pl.ds(c * C, C)] = res[k + 1]
            return carry

        lax.fori_loop(0, n_chunks, chunk_body, 0)
        pltpu.sync_copy(pos_o, pos_h.at[pl.ds(base, Bw)])
        pltpu.sync_copy(neg_o, negd_h.at[wid])

    return run(x, y, neg_flat, WI, WO, seq_table, T)


def _tc_loss(pos2d, neg2d, B):
    def body(p_ref, n_ref, o_ref):
        p = p_ref[...]
        n = n_ref[...]
        ls_p = jnp.minimum(p, 0.0) - jnp.log(1.0 + jnp.exp(-jnp.abs(p)))
        ls_n = jnp.minimum(-n, 0.0) - jnp.log(1.0 + jnp.exp(-jnp.abs(n)))
        o_ref[0, 0] = -(jnp.sum(ls_p) / B) - jnp.sum(ls_n)

    return pl.pallas_call(
        body,
        out_shape=jax.ShapeDtypeStruct((1, 1), jnp.float32),
        out_specs=pl.BlockSpec(memory_space=pltpu.SMEM),
    )(pos2d, neg2d)


def kernel(x, y, neg_lookup, WI, WO, seq_table, T):
    B = x.shape[0]
    K = neg_lookup.shape[1]
    D = WI.shape[1]
    info = plsc.get_sparse_core_info()
    NC, NS = info.num_cores, info.num_subcores
    pos, negd = _sc_dots(
        x.astype(jnp.int32), y.astype(jnp.int32),
        neg_lookup.astype(jnp.int32).reshape(B * K),
        WI, WO, seq_table, T, B=B, K=K, D=D, NC=NC, NS=NS)
    n_all = B * K
    loss = _tc_loss(pos.reshape(B // 128, 128), negd.reshape(n_all // 128, 128), B)
    return loss[0, 0]


# speedup vs baseline: 1.2046x; 1.2046x over previous
import functools

import jax
import jax.numpy as jnp
from jax import lax
from jax.experimental import pallas as pl
from jax.experimental.pallas import tpu as pltpu
from jax.experimental.pallas import tpu_sc as plsc

_LANES = 16
_C = 16


def _sc_dots(x3, y3, neg3, WI, WO, seq_table, T, *, B, K, D, NC, NS):
    NW = NC * NS
    Bw = B // NW
    C = _C
    n_chunks = Bw // C
    JV = D // _LANES

    mesh = plsc.VectorSubcoreMesh(core_axis_name="c", subcore_axis_name="s")

    row_buf = lambda n: pltpu.VMEM((n, D), jnp.float32)
    buf_set = [row_buf(C), row_buf(C), row_buf(C), row_buf(C * K), row_buf(C * K),
               pltpu.SemaphoreType.DMA]

    @functools.partial(
        pl.kernel,
        mesh=mesh,
        compiler_params=pltpu.CompilerParams(needs_layout_passes=False),
        out_type=[
            jax.ShapeDtypeStruct((B,), jnp.float32),
            jax.ShapeDtypeStruct((NW, K, Bw), jnp.float32),
        ],
        scratch_types=[
            pltpu.VMEM((n_chunks, C), jnp.int32),
            pltpu.VMEM((n_chunks, C), jnp.int32),
            pltpu.VMEM((n_chunks, C * K), jnp.int32),
            buf_set, buf_set,
            pltpu.VMEM((C, D), jnp.float32),
            pltpu.VMEM((D,), jnp.float32),
            pltpu.VMEM((Bw,), jnp.float32),
            pltpu.VMEM((K, Bw), jnp.float32),
        ],
    )
    def run(x_h, y_h, neg_h, wi_h, wo_h, seq_h, t_h, pos_h, negd_h,
            ix2, iy2, in2, buf0, buf1, vit_v, t_v, pos_o, neg_o):
        wid = lax.axis_index("s") * NC + lax.axis_index("c")
        base = wid * Bw
        pltpu.sync_copy(x_h.at[wid], ix2)
        pltpu.sync_copy(y_h.at[wid], iy2)
        pltpu.sync_copy(neg_h.at[wid], in2)
        pltpu.sync_copy(t_h, t_v)
        tj = [t_v[pl.ds(j * _LANES, _LANES)] for j in range(JV)]
        lane = lax.broadcasted_iota(jnp.int32, (_LANES,), 0)
        rowk = [lane * K + k for k in range(K)]
        bufs = (buf0, buf1)

        def gathers(c, buf):
            vi_b, woy_b, seqy_b, won_b, seqn_b, sem = buf
            return [
                pltpu.make_async_copy(wi_h.at[ix2.at[c]], vi_b, sem),
                pltpu.make_async_copy(wo_h.at[iy2.at[c]], woy_b, sem),
                pltpu.make_async_copy(seq_h.at[iy2.at[c]], seqy_b, sem),
                pltpu.make_async_copy(wo_h.at[in2.at[c]], won_b, sem),
                pltpu.make_async_copy(seq_h.at[in2.at[c]], seqn_b, sem),
            ]

        def issue(c, buf):
            for cp in gathers(c, buf):
                cp.start()

        def drain(c, buf):
            for cp in gathers(c, buf):
                cp.wait()

        def compute(c, buf):
            vi_b, woy_b, seqy_b, won_b, seqn_b, _ = buf

            def pre_body(g, acc):
                for j in range(JV):
                    sl = pl.ds(j * _LANES, _LANES)
                    vit_v[g, sl] = vi_b[g, sl] * tj[j]
                return acc
            lax.fori_loop(0, C, pre_body, 0)

            def d_body(i, accs):
                for dd in range(4):
                    d = i * 4 + dd
                    col = jnp.zeros((_LANES,), jnp.int32) + d
                    vi_c = plsc.load_gather(vi_b, [lane, col])
                    vit_c = plsc.load_gather(vit_v, [lane, col])
                    a0 = (accs[0]
                          + vi_c * plsc.load_gather(woy_b, [lane, col])
                          + vit_c * plsc.load_gather(seqy_b, [lane, col]))
                    new = [a0]
                    for k in range(K):
                        new.append(accs[k + 1]
                                   + vi_c * plsc.load_gather(won_b, [rowk[k], col])
                                   + vit_c * plsc.load_gather(seqn_b, [rowk[k], col]))
                    accs = tuple(new)
                return accs

            init = tuple(jnp.zeros((_LANES,), jnp.float32) for _ in range(K + 1))
            res = lax.fori_loop(0, D // 4, d_body, init)
            pos_o[pl.ds(c * C, C)] = res[0]
            for k in range(K):
                neg_o[k, pl.ds(c * C, C)] = res[k + 1]

        issue(0, bufs[0])
        issue(1, bufs[1])

        def pair_body(i, carry):
            for p in range(2):
                c = 2 * i + p
                drain(c, bufs[p])
                compute(c, bufs[p])
                nxt = c + 2

                @pl.when(nxt < n_chunks)
                def _():
                    issue(nxt, bufs[p])
            return carry

        lax.fori_loop(0, n_chunks // 2, pair_body, 0)
        pltpu.sync_copy(pos_o, pos_h.at[pl.ds(base, Bw)])
        pltpu.sync_copy(neg_o, negd_h.at[wid])

    return run(x3, y3, neg3, WI, WO, seq_table, T)


def _tc_loss(pos2d, neg2d, B):
    def body(p_ref, n_ref, o_ref):
        p = p_ref[...]
        n = n_ref[...]
        ls_p = jnp.minimum(p, 0.0) - jnp.log(1.0 + jnp.exp(-jnp.abs(p)))
        ls_n = jnp.minimum(-n, 0.0) - jnp.log(1.0 + jnp.exp(-jnp.abs(n)))
        o_ref[0, 0] = -(jnp.sum(ls_p) / B) - jnp.sum(ls_n)

    return pl.pallas_call(
        body,
        out_shape=jax.ShapeDtypeStruct((1, 1), jnp.float32),
        out_specs=pl.BlockSpec(memory_space=pltpu.SMEM),
    )(pos2d, neg2d)


def kernel(x, y, neg_lookup, WI, WO, seq_table, T):
    B = x.shape[0]
    K = neg_lookup.shape[1]
    D = WI.shape[1]
    info = plsc.get_sparse_core_info()
    NC, NS = info.num_cores, info.num_subcores
    NW = NC * NS
    n_chunks = B // NW // _C
    pos, negd = _sc_dots(
        x.astype(jnp.int32).reshape(NW, n_chunks, _C),
        y.astype(jnp.int32).reshape(NW, n_chunks, _C),
        neg_lookup.astype(jnp.int32).reshape(NW, n_chunks, _C * K),
        WI, WO, seq_table, T, B=B, K=K, D=D, NC=NC, NS=NS)
    n_all = B * K
    loss = _tc_loss(pos.reshape(B // 128, 128), negd.reshape(n_all // 128, 128), B)
    return loss[0, 0]


# speedup vs baseline: 6.4771x; 5.3768x over previous
import functools

import jax
import jax.numpy as jnp
from jax import lax
from jax.experimental import pallas as pl
from jax.experimental.pallas import tpu as pltpu
from jax.experimental.pallas import tpu_sc as plsc

_LANES = 16
_C = 16


def _sc_dots(x3, yn3, WI, WO, seq_table, T, *, B, K, D, NC, NS):
    NW = NC * NS
    Bw = B // NW
    C = _C
    R = C * (K + 1)
    n_chunks = Bw // C
    JV = D // _LANES

    mesh = plsc.VectorSubcoreMesh(core_axis_name="c", subcore_axis_name="s")

    buf_set = [pltpu.VMEM((C, D), jnp.float32),
               pltpu.VMEM((R, D), jnp.float32),
               pltpu.VMEM((R, D), jnp.float32),
               pltpu.SemaphoreType.DMA]

    @functools.partial(
        pl.kernel,
        mesh=mesh,
        compiler_params=pltpu.CompilerParams(needs_layout_passes=False),
        out_type=[
            jax.ShapeDtypeStruct((B,), jnp.float32),
            jax.ShapeDtypeStruct((NW, K, Bw), jnp.float32),
        ],
        scratch_types=[
            pltpu.VMEM((n_chunks, C), jnp.int32),
            pltpu.VMEM((n_chunks, R), jnp.int32),
            buf_set, buf_set,
            pltpu.VMEM((D,), jnp.float32),
            pltpu.VMEM((Bw,), jnp.float32),
            pltpu.VMEM((K, Bw), jnp.float32),
        ],
    )
    def run(x_h, yn_h, wi_h, wo_h, seq_h, t_h, pos_h, negd_h,
            ix2, iyn2, buf0, buf1, t_v, pos_o, neg_o):
        wid = lax.axis_index("s") * NC + lax.axis_index("c")
        base = wid * Bw
        pltpu.sync_copy(x_h.at[wid], ix2)
        pltpu.sync_copy(yn_h.at[wid], iyn2)
        pltpu.sync_copy(t_h, t_v)
        tj = [t_v[pl.ds(j * _LANES, _LANES)] for j in range(JV)]
        lane = lax.broadcasted_iota(jnp.int32, (_LANES,), 0)
        bufs = (buf0, buf1)

        def gathers(c, buf):
            vi_b, wo_b, seq_b, sem = buf
            return [
                pltpu.make_async_copy(wi_h.at[ix2.at[c]], vi_b, sem),
                pltpu.make_async_copy(wo_h.at[iyn2.at[c]], wo_b, sem),
                pltpu.make_async_copy(seq_h.at[iyn2.at[c]], seq_b, sem),
            ]

        def issue(c, buf):
            for cp in gathers(c, buf):
                cp.start()

        def drain(c, buf):
            for cp in gathers(c, buf):
                cp.wait()

        def compute(c, buf):
            vi_b, wo_b, seq_b, _ = buf

            def token_body(g, vecs):
                r0 = g * (K + 1)
                accs = [jnp.zeros((_LANES,), jnp.float32) for _ in range(K + 1)]
                for j in range(JV):
                    sl = pl.ds(j * _LANES, _LANES)
                    vij = vi_b[g, sl]
                    vitj = vij * tj[j]
                    for k in range(K + 1):
                        accs[k] = (accs[k] + vij * wo_b[r0 + k, sl]
                                   + vitj * seq_b[r0 + k, sl])
                m = lane == g
                return tuple(jnp.where(m, jnp.sum(accs[i]), vecs[i])
                             for i in range(K + 1))

            init = tuple(jnp.zeros((_LANES,), jnp.float32) for _ in range(K + 1))
            res = lax.fori_loop(0, C, token_body, init)
            pos_o[pl.ds(c * C, C)] = res[0]
            for k in range(K):
                neg_o[k, pl.ds(c * C, C)] = res[k + 1]

        issue(0, bufs[0])
        issue(1, bufs[1])

        def pair_body(i, carry):
            for p in range(2):
                c = 2 * i + p
                drain(c, bufs[p])
                compute(c, bufs[p])
                nxt = c + 2

                @pl.when(nxt < n_chunks)
                def _():
                    issue(nxt, bufs[p])
            return carry

        lax.fori_loop(0, n_chunks // 2, pair_body, 0)
        pltpu.sync_copy(pos_o, pos_h.at[pl.ds(base, Bw)])
        pltpu.sync_copy(neg_o, negd_h.at[wid])

    return run(x3, yn3, WI, WO, seq_table, T)


def _tc_loss(pos2d, neg2d, B):
    def body(p_ref, n_ref, o_ref):
        p = p_ref[...]
        n = n_ref[...]
        ls_p = jnp.minimum(p, 0.0) - jnp.log(1.0 + jnp.exp(-jnp.abs(p)))
        ls_n = jnp.minimum(-n, 0.0) - jnp.log(1.0 + jnp.exp(-jnp.abs(n)))
        o_ref[0, 0] = -(jnp.sum(ls_p) / B) - jnp.sum(ls_n)

    return pl.pallas_call(
        body,
        out_shape=jax.ShapeDtypeStruct((1, 1), jnp.float32),
        out_specs=pl.BlockSpec(memory_space=pltpu.SMEM),
    )(pos2d, neg2d)


def kernel(x, y, neg_lookup, WI, WO, seq_table, T):
    B = x.shape[0]
    K = neg_lookup.shape[1]
    D = WI.shape[1]
    info = plsc.get_sparse_core_info()
    NC, NS = info.num_cores, info.num_subcores
    NW = NC * NS
    n_chunks = B // NW // _C
    yn = jnp.concatenate(
        [y.astype(jnp.int32)[:, None], neg_lookup.astype(jnp.int32)], axis=1)
    pos, negd = _sc_dots(
        x.astype(jnp.int32).reshape(NW, n_chunks, _C),
        yn.reshape(NW, n_chunks, _C * (K + 1)),
        WI, WO, seq_table, T, B=B, K=K, D=D, NC=NC, NS=NS)
    n_all = B * K
    loss = _tc_loss(pos.reshape(B // 128, 128), negd.reshape(n_all // 128, 128), B)
    return loss[0, 0]


# speedup vs baseline: 7.0916x; 1.0949x over previous
import functools

import jax
import jax.numpy as jnp
from jax import lax
from jax.experimental import pallas as pl
from jax.experimental.pallas import tpu as pltpu
from jax.experimental.pallas import tpu_sc as plsc

_LANES = 16
_C = 16


def _sc_dots(x3, yn3, WI, WO, seq_table, T, *, B, K, D, NC, NS):
    NW = NC * NS
    Bw = B // NW
    C = _C
    R = C * (K + 1)
    n_chunks = Bw // C
    JV = D // _LANES

    mesh = plsc.VectorSubcoreMesh(core_axis_name="c", subcore_axis_name="s")

    buf_set = [pltpu.VMEM((C, D), jnp.float32),
               pltpu.VMEM((R, D), jnp.float32),
               pltpu.VMEM((R, D), jnp.float32),
               pltpu.SemaphoreType.DMA]

    @functools.partial(
        pl.kernel,
        mesh=mesh,
        compiler_params=pltpu.CompilerParams(needs_layout_passes=False),
        out_type=[
            jax.ShapeDtypeStruct((B,), jnp.float32),
            jax.ShapeDtypeStruct((NW, K, Bw), jnp.float32),
        ],
        scratch_types=[
            pltpu.VMEM((n_chunks, C), jnp.int32),
            pltpu.VMEM((n_chunks, R), jnp.int32),
            buf_set, buf_set, buf_set,
            pltpu.VMEM((D,), jnp.float32),
            pltpu.VMEM((Bw,), jnp.float32),
            pltpu.VMEM((K, Bw), jnp.float32),
        ],
    )
    def run(x_h, yn_h, wi_h, wo_h, seq_h, t_h, pos_h, negd_h,
            ix2, iyn2, buf0, buf1, buf2, t_v, pos_o, neg_o):
        wid = lax.axis_index("s") * NC + lax.axis_index("c")
        base = wid * Bw
        pltpu.sync_copy(x_h.at[wid], ix2)
        pltpu.sync_copy(yn_h.at[wid], iyn2)
        pltpu.sync_copy(t_h, t_v)
        tj = [t_v[pl.ds(j * _LANES, _LANES)] for j in range(JV)]
        lane = lax.broadcasted_iota(jnp.int32, (_LANES,), 0)
        bufs = (buf0, buf1, buf2)
        NB = len(bufs)

        def gathers(c, buf):
            vi_b, wo_b, seq_b, sem = buf
            return [
                pltpu.make_async_copy(wi_h.at[ix2.at[c]], vi_b, sem),
                pltpu.make_async_copy(wo_h.at[iyn2.at[c]], wo_b, sem),
                pltpu.make_async_copy(seq_h.at[iyn2.at[c]], seq_b, sem),
            ]

        def issue(c, buf):
            for cp in gathers(c, buf):
                cp.start()

        def drain(c, buf):
            for cp in gathers(c, buf):
                cp.wait()

        def compute(c, buf):
            vi_b, wo_b, seq_b, _ = buf

            def token_body(g, vecs):
                r0 = g * (K + 1)
                accs = [jnp.zeros((_LANES,), jnp.float32) for _ in range(K + 1)]
                for j in range(JV):
                    sl = pl.ds(j * _LANES, _LANES)
                    vij = vi_b[g, sl]
                    vitj = vij * tj[j]
                    for k in range(K + 1):
                        accs[k] = (accs[k] + vij * wo_b[r0 + k, sl]
                                   + vitj * seq_b[r0 + k, sl])
                m = lane == g
                return tuple(jnp.where(m, jnp.sum(accs[i]), vecs[i])
                             for i in range(K + 1))

            init = tuple(jnp.zeros((_LANES,), jnp.float32) for _ in range(K + 1))
            res = lax.fori_loop(0, C, token_body, init)
            pos_o[pl.ds(c * C, C)] = res[0]
            for k in range(K):
                neg_o[k, pl.ds(c * C, C)] = res[k + 1]

        for p in range(NB):
            issue(p, bufs[p])

        def ring_body(i, carry):
            for p in range(NB):
                c = NB * i + p
                drain(c, bufs[p])
                compute(c, bufs[p])
                nxt = c + NB

                @pl.when(nxt < n_chunks)
                def _():
                    issue(nxt, bufs[p])
            return carry

        main = n_chunks // NB
        lax.fori_loop(0, main, ring_body, 0)
        for c in range(NB * main, n_chunks):
            drain(c, bufs[c % NB])
            compute(c, bufs[c % NB])
        pltpu.sync_copy(pos_o, pos_h.at[pl.ds(base, Bw)])
        pltpu.sync_copy(neg_o, negd_h.at[wid])

    return run(x3, yn3, WI, WO, seq_table, T)


def _tc_loss(pos2d, neg2d, B):
    def body(p_ref, n_ref, o_ref):
        p = p_ref[...]
        n = n_ref[...]
        ls_p = jnp.minimum(p, 0.0) - jnp.log(1.0 + jnp.exp(-jnp.abs(p)))
        ls_n = jnp.minimum(-n, 0.0) - jnp.log(1.0 + jnp.exp(-jnp.abs(n)))
        o_ref[0, 0] = -(jnp.sum(ls_p) / B) - jnp.sum(ls_n)

    return pl.pallas_call(
        body,
        out_shape=jax.ShapeDtypeStruct((1, 1), jnp.float32),
        out_specs=pl.BlockSpec(memory_space=pltpu.SMEM),
    )(pos2d, neg2d)


def kernel(x, y, neg_lookup, WI, WO, seq_table, T):
    B = x.shape[0]
    K = neg_lookup.shape[1]
    D = WI.shape[1]
    info = plsc.get_sparse_core_info()
    NC, NS = info.num_cores, info.num_subcores
    NW = NC * NS
    n_chunks = B // NW // _C
    yn = jnp.concatenate(
        [y.astype(jnp.int32)[:, None], neg_lookup.astype(jnp.int32)], axis=1)
    pos, negd = _sc_dots(
        x.astype(jnp.int32).reshape(NW, n_chunks, _C),
        yn.reshape(NW, n_chunks, _C * (K + 1)),
        WI, WO, seq_table, T, B=B, K=K, D=D, NC=NC, NS=NS)
    n_all = B * K
    loss = _tc_loss(pos.reshape(B // 128, 128), negd.reshape(n_all // 128, 128), B)
    return loss[0, 0]
